# baseline (speedup 1.0000x reference)
import functools

import jax
import jax.numpy as jnp
from jax import lax
from jax.experimental import pallas as pl
from jax.experimental.pallas import tpu as pltpu
from jax.experimental.pallas import tpu_sc as plsc

B = 16384
D = 64
L = 16
NC = 2
NS = 16
NW = NC * NS
BPW = B // NW
CHUNK = 128
NCHUNK = BPW // CHUNK
NACC = 4

_mesh = plsc.VectorSubcoreMesh(core_axis_name="c", subcore_axis_name="s")


@functools.partial(
    pl.kernel,
    out_type=jax.ShapeDtypeStruct((B,), jnp.float32),
    mesh=_mesh,
    compiler_params=pltpu.CompilerParams(
        needs_layout_passes=False, use_tc_tiling_on_sc=False),
    scratch_types=[
        pltpu.VMEM((NCHUNK, CHUNK), jnp.int32),
        pltpu.VMEM((NCHUNK, CHUNK), jnp.int32),
        pltpu.VMEM((CHUNK, D), jnp.float32),
        pltpu.VMEM((CHUNK, D), jnp.float32),
        pltpu.VMEM((BPW,), jnp.float32),
        pltpu.VMEM((D * L + L,), jnp.float32),
        pltpu.SemaphoreType.DMA,
        pltpu.SemaphoreType.DMA,
    ],
)
def _gmf_sc(uidx_hbm, vidx_hbm, ut_hbm, it_hbm, wb_hbm, out_hbm,
            uidx_v, vidx_v, urows, vrows, outv, wv, sem_u, sem_v):
    wid = lax.axis_index("s") * NC + lax.axis_index("c")
    base = wid * BPW

    pltpu.sync_copy(uidx_hbm.at[wid], uidx_v)
    pltpu.sync_copy(vidx_hbm.at[wid], vidx_v)
    pltpu.sync_copy(wb_hbm, wv)

    bvec = wv[pl.ds(D * L, L)]
    lane = lax.iota(jnp.int32, L)

    for j in range(NCHUNK):
        cu = pltpu.async_copy(ut_hbm.at[uidx_v.at[j]], urows, sem_u)
        cv = pltpu.async_copy(it_hbm.at[vidx_v.at[j]], vrows, sem_v)
        cu.wait()
        cv.wait()

        @plsc.parallel_loop(0, CHUNK // L, 1, unroll=2)
        def group_body(g, j=j):
            row = g * L + lane
            accs = []
            for d in range(NACC):
                col = jnp.full((L,), d, dtype=jnp.int32)
                accs.append(plsc.load_gather(urows, [row, col])
                            * plsc.load_gather(vrows, [row, col])
                            * wv[pl.ds(d * L, L)])
            for d in range(NACC, D):
                col = jnp.full((L,), d, dtype=jnp.int32)
                accs[d % NACC] += (plsc.load_gather(urows, [row, col])
                                   * plsc.load_gather(vrows, [row, col])
                                   * wv[pl.ds(d * L, L)])
            acc = (accs[0] + accs[1]) + (accs[2] + accs[3]) + bvec
            outv[pl.ds(j * CHUNK + g * L, L)] = 1.0 / (1.0 + jnp.exp(-acc))

    pltpu.sync_copy(outv, out_hbm.at[pl.ds(base, BPW)])


def kernel(input, user_table, item_table, W, b):
    idx = input.astype(jnp.int32)
    uidx = idx[:, 0].reshape(NW, NCHUNK, CHUNK)
    vidx = idx[:, 1].reshape(NW, NCHUNK, CHUNK)
    wb = jnp.concatenate([
        jnp.broadcast_to(W.reshape(D, 1), (D, L)).reshape(D * L),
        jnp.broadcast_to(b, (L,)),
    ])
    return _gmf_sc(uidx, vidx, user_table, item_table, wb)


# speedup vs baseline: 1.2448x; 1.2448x over previous
import functools

import jax
import jax.numpy as jnp
from jax import lax
from jax.experimental import pallas as pl
from jax.experimental.pallas import tpu as pltpu
from jax.experimental.pallas import tpu_sc as plsc

B = 16384
D = 64
L = 16
NC = 2
NS = 16
NW = NC * NS
BPW = B // NW
CHUNK = 128
NCHUNK = BPW // CHUNK

_mesh = plsc.VectorSubcoreMesh(core_axis_name="c", subcore_axis_name="s")


@functools.partial(
    pl.kernel,
    out_type=jax.ShapeDtypeStruct((B,), jnp.float32),
    mesh=_mesh,
    compiler_params=pltpu.CompilerParams(
        needs_layout_passes=False, use_tc_tiling_on_sc=False),
    scratch_types=[
        pltpu.VMEM((BPW,), jnp.int32),
        pltpu.VMEM((BPW,), jnp.int32),
        pltpu.VMEM((CHUNK, D), jnp.float32),
        pltpu.VMEM((CHUNK, D), jnp.float32),
        pltpu.VMEM((BPW,), jnp.float32),
        pltpu.VMEM((L * L,), jnp.float32),
        pltpu.VMEM((D + L,), jnp.float32),
        pltpu.SemaphoreType.DMA,
        pltpu.SemaphoreType.DMA,
    ],
)
def _gmf_sc(uidx_hbm, vidx_hbm, ut_hbm, it_hbm, wb_hbm, out_hbm,
            uidx_v, vidx_v, urows, vrows, outv, mat, wv, sem_u, sem_v):
    wid = lax.axis_index("s") * NC + lax.axis_index("c")
    base = wid * BPW

    pltpu.sync_copy(uidx_hbm.at[pl.ds(base, BPW)], uidx_v)
    pltpu.sync_copy(vidx_hbm.at[pl.ds(base, BPW)], vidx_v)
    pltpu.sync_copy(wb_hbm, wv)

    w = [wv[pl.ds(c * L, L)] for c in range(D // L)]
    bvec = wv[pl.ds(D, L)]
    col_base = lax.iota(jnp.int32, L) * L
    idxcol = [col_base + l for l in range(L)]

    for j in range(NCHUNK):
        cu = pltpu.async_copy(ut_hbm.at[uidx_v.at[pl.ds(j * CHUNK, CHUNK)]],
                              urows, sem_u)
        cv = pltpu.async_copy(it_hbm.at[vidx_v.at[pl.ds(j * CHUNK, CHUNK)]],
                              vrows, sem_v)
        cu.wait()
        cv.wait()

        def group_body(g, carry, j=j):
            i0 = g * L
            for r in range(L):
                acc = (urows[i0 + r, pl.ds(0, L)]
                       * vrows[i0 + r, pl.ds(0, L)]) * w[0]
                for c in range(1, D // L):
                    acc += (urows[i0 + r, pl.ds(c * L, L)]
                            * vrows[i0 + r, pl.ds(c * L, L)]) * w[c]
                mat[pl.ds(r * L, L)] = acc
            cols = [plsc.load_gather(mat, [idxcol[l]]) for l in range(L)]
            while len(cols) > 1:
                cols = [cols[i] + cols[i + 1] for i in range(0, len(cols), 2)]
            colsum = cols[0] + bvec
            outv[pl.ds(j * CHUNK + i0, L)] = 1.0 / (1.0 + jnp.exp(-colsum))
            return carry

        lax.fori_loop(0, CHUNK // L, group_body, 0)

    pltpu.sync_copy(outv, out_hbm.at[pl.ds(base, BPW)])


def kernel(input, user_table, item_table, W, b):
    idx = input.astype(jnp.int32)
    wb = jnp.concatenate([W.reshape(D), jnp.broadcast_to(b, (L,))])
    return _gmf_sc(idx[:, 0], idx[:, 1], user_table, item_table, wb)
